# Initial kernel scaffold; baseline (speedup 1.0000x reference)
import jax
import jax.numpy as jnp
from jax.experimental import pallas as pl


def kernel(x, table):
    raise NotImplementedError("write your pallas kernel here")


# speedup vs baseline: 1.0558x; 1.0558x over previous
import functools

import numpy as np
import jax
import jax.numpy as jnp
from jax import lax
from jax.experimental import pallas as pl
from jax.experimental.pallas import tpu as pltpu
from jax.experimental.pallas import tpu_sc as plsc

VOCAB = 100000
D_MODEL = 1024
MAX_LENGTH = 2048
SCALE = float(np.sqrt(D_MODEL))


def _positional_encoding(length, depth):
    half = depth // 2
    positions = np.arange(length)[:, np.newaxis]
    depths = np.arange(half)[np.newaxis, :] / half
    angle_rates = 1 / 10000**depths
    angle_rads = positions * angle_rates
    return np.concatenate(
        [np.sin(angle_rads), np.cos(angle_rads)], axis=-1
    ).astype(np.float32)


_INFO = plsc.get_sparse_core_info()
_NC, _NS, _L = _INFO.num_cores, _INFO.num_subcores, _INFO.num_lanes
_NW = _NC * _NS

_B = 4
_LEN = 2048
_PW = _LEN // _NW
_CH = 32
_SUB = _PW // _CH
_VREGS = _CH * D_MODEL // 16


def _body(x_hbm, pos_hbm, table_hbm, out_hbm,
          pos_v, row_v0, row_v1, idx_v0, idx_v1,
          gsem0, gsem1, osem0, osem1):
    wid = lax.axis_index("s") * _NC + lax.axis_index("c")
    base = wid * _PW

    row_v = (row_v0, row_v1)
    idx_v = (idx_v0, idx_v1)
    gsem = (gsem0, gsem1)
    osem = (osem0, osem1)

    rounds = [(s, b) for s in range(_SUB) for b in range(_B)]
    pend_out = [None, None]

    def start_gather(k):
        s, b = rounds[k]
        slot = k % 2
        if pend_out[slot] is not None:
            pend_out[slot].wait()
            pend_out[slot] = None
        pltpu.sync_copy(x_hbm.at[b, pl.ds(base + s * _CH, _CH)], idx_v[slot])
        return pltpu.async_copy(table_hbm.at[idx_v[slot]], row_v[slot],
                                gsem[slot])

    def compute(slot):
        row = row_v[slot]

        def inner(i, _):
            r = i // 16
            c = (i % 16) * 64
            for u in range(4):
                sl = pl.ds(c + u * 16, 16)
                row[r, sl] = row[r, sl] * SCALE + pos_v[r, sl]
            return _

        lax.fori_loop(0, _VREGS // 4, inner, None)

    g = start_gather(0)
    for k in range(len(rounds)):
        s, b = rounds[k]
        slot = k % 2
        if b == 0:
            pltpu.sync_copy(pos_hbm.at[pl.ds(base + s * _CH, _CH), :], pos_v)
        g.wait()
        gn = start_gather(k + 1) if k + 1 < len(rounds) else None
        compute(slot)
        pend_out[slot] = pltpu.async_copy(
            row_v[slot], out_hbm.at[b, pl.ds(base + s * _CH, _CH), :],
            osem[slot])
        g = gn
    for p in pend_out:
        if p is not None:
            p.wait()


_sc_call = pl.kernel(
    _body,
    out_type=jax.ShapeDtypeStruct((_B, _LEN, D_MODEL), jnp.float32),
    mesh=plsc.VectorSubcoreMesh(core_axis_name="c", subcore_axis_name="s"),
    scratch_types=[
        pltpu.VMEM((_CH, D_MODEL), jnp.float32),
        pltpu.VMEM((_CH, D_MODEL), jnp.float32),
        pltpu.VMEM((_CH, D_MODEL), jnp.float32),
        pltpu.VMEM((_CH,), jnp.int32),
        pltpu.VMEM((_CH,), jnp.int32),
        pltpu.SemaphoreType.DMA,
        pltpu.SemaphoreType.DMA,
        pltpu.SemaphoreType.DMA,
        pltpu.SemaphoreType.DMA,
    ],
)

_POS = _positional_encoding(MAX_LENGTH, D_MODEL)[:_LEN]


@jax.jit
def kernel(x, table):
    pos = jnp.asarray(_POS)
    return _sc_call(x.astype(jnp.int32), pos, table)


# speedup vs baseline: 1.1322x; 1.0724x over previous
import functools

import numpy as np
import jax
import jax.numpy as jnp
from jax import lax
from jax.experimental import pallas as pl
from jax.experimental.pallas import tpu as pltpu
from jax.experimental.pallas import tpu_sc as plsc

VOCAB = 100000
D_MODEL = 1024
MAX_LENGTH = 2048
SCALE = float(np.sqrt(D_MODEL))


def _positional_encoding(length, depth):
    half = depth // 2
    positions = np.arange(length)[:, np.newaxis]
    depths = np.arange(half)[np.newaxis, :] / half
    angle_rates = 1 / 10000**depths
    angle_rads = positions * angle_rates
    return np.concatenate(
        [np.sin(angle_rads), np.cos(angle_rads)], axis=-1
    ).astype(np.float32)


_INFO = plsc.get_sparse_core_info()
_NC, _NS, _L = _INFO.num_cores, _INFO.num_subcores, _INFO.num_lanes
_NW = _NC * _NS

_B = 4
_LEN = 2048
_PW = _LEN // _NW
_CH = 32
_SUB = _PW // _CH
_VREGS = _CH * D_MODEL // 16


def _body(x_hbm, pos_hbm, table_hbm, out_hbm,
          pos_v, row_v0, row_v1, idx_v0, idx_v1,
          gsem0, gsem1, osem0, osem1):
    wid = lax.axis_index("s") * _NC + lax.axis_index("c")
    base = wid * _PW

    row_v = (row_v0, row_v1)
    idx_v = (idx_v0, idx_v1)
    gsem = (gsem0, gsem1)
    osem = (osem0, osem1)

    rounds = [(s, b) for s in range(_SUB) for b in range(_B)]
    pend_out = [None, None]

    def start_gather(k):
        s, b = rounds[k]
        slot = k % 2
        if pend_out[slot] is not None:
            pend_out[slot].wait()
            pend_out[slot] = None
        pltpu.sync_copy(x_hbm.at[b, pl.ds(base + s * _CH, _CH)], idx_v[slot])
        return pltpu.async_copy(table_hbm.at[idx_v[slot]], row_v[slot],
                                gsem[slot])

    def compute(slot):
        row = row_v[slot]

        @plsc.parallel_loop(0, _VREGS, 1, unroll=8)
        def _(i):
            r = i // 64
            sl = pl.ds((i % 64) * 16, 16)
            row[r, sl] = row[r, sl] * SCALE + pos_v[r, sl]

    g = start_gather(0)
    for k in range(len(rounds)):
        s, b = rounds[k]
        slot = k % 2
        if b == 0:
            pltpu.sync_copy(pos_hbm.at[pl.ds(base + s * _CH, _CH), :], pos_v)
        g.wait()
        gn = start_gather(k + 1) if k + 1 < len(rounds) else None
        compute(slot)
        pend_out[slot] = pltpu.async_copy(
            row_v[slot], out_hbm.at[b, pl.ds(base + s * _CH, _CH), :],
            osem[slot])
        g = gn
    for p in pend_out:
        if p is not None:
            p.wait()


_sc_call = pl.kernel(
    _body,
    out_type=jax.ShapeDtypeStruct((_B, _LEN, D_MODEL), jnp.float32),
    mesh=plsc.VectorSubcoreMesh(core_axis_name="c", subcore_axis_name="s"),
    scratch_types=[
        pltpu.VMEM((_CH, D_MODEL), jnp.float32),
        pltpu.VMEM((_CH, D_MODEL), jnp.float32),
        pltpu.VMEM((_CH, D_MODEL), jnp.float32),
        pltpu.VMEM((_CH,), jnp.int32),
        pltpu.VMEM((_CH,), jnp.int32),
        pltpu.SemaphoreType.DMA,
        pltpu.SemaphoreType.DMA,
        pltpu.SemaphoreType.DMA,
        pltpu.SemaphoreType.DMA,
    ],
)

_POS = _positional_encoding(MAX_LENGTH, D_MODEL)[:_LEN]


@jax.jit
def kernel(x, table):
    pos = jnp.asarray(_POS)
    return _sc_call(x.astype(jnp.int32), pos, table)
